# baseline (speedup 1.0000x reference)
import jax
import jax.numpy as jnp
from jax.experimental import pallas as pl
from jax.experimental.pallas import tpu as pltpu

_N = 10000
_A = 32
_NODE_DIM = 256
_ATTR_DIM = 512
_HIDDEN = 128
_B = 5000
_NB = _N // _B


def _body(qa_ref, v_ref, emb_ref, win_ref, bin_ref, wa_ref, ua_ref, ba_ref,
          wv_ref, uv_ref, bv_ref, wout_ref, bout_ref, out_ref,
          g1_scr, ha0_scr, c_scr, vsum_scr, m0_scr):
    p = pl.program_id(0)
    i = pl.program_id(1)

    @pl.when((p == 0) & (i == 0))
    def _init():
        oneT = (jax.lax.broadcasted_iota(jnp.int32, (_ATTR_DIM, _A), 0)
                == qa_ref[:, :]).astype(jnp.float32)
        ha0 = jax.lax.dot_general(oneT, emb_ref[:, :], (((0,), (0,)), ((), ())),
                                  preferred_element_type=jnp.float32)
        ha0_scr[:, :] = ha0
        sa0 = jnp.sum(ha0, axis=0, keepdims=True) / float(_A)
        c0 = (jnp.dot(sa0.astype(jnp.bfloat16), wv_ref[0], preferred_element_type=jnp.float32)
              + bv_ref[0])
        m0_scr[:, :] = jnp.dot(win_ref[:, :], uv_ref[0],
                               preferred_element_type=jnp.float32
                               ).astype(jnp.bfloat16)
        c_scr[0:1, :] = (jnp.dot(bin_ref[:, :], uv_ref[0],
                                 preferred_element_type=jnp.float32) + c0)
        vsum_scr[:, :] = jnp.zeros((1, _NODE_DIM), jnp.float32)

    @pl.when(p == 0)
    def _pass0():
        x = v_ref[:, :]
        vsum_scr[:, :] += jnp.sum(x, axis=0, keepdims=True)
        h1 = jnp.maximum(
            jnp.dot(x.astype(jnp.bfloat16), m0_scr[:, :],
                    preferred_element_type=jnp.float32)
            + c_scr[0:1, :], 0.0)
        g1_scr[pl.ds(i * _B, _B), :] = jnp.dot(
            h1.astype(jnp.bfloat16), uv_ref[1],
            preferred_element_type=jnp.float32)

    @pl.when((p == 1) & (i == 0))
    def _mid():
        sv0 = (jnp.dot((vsum_scr[:, :] / float(_N)).astype(jnp.bfloat16),
                       win_ref[:, :],
                       preferred_element_type=jnp.float32) + bin_ref[:, :])
        ha1 = jnp.maximum(
            jnp.dot(sv0.astype(jnp.bfloat16), wa_ref[0],
                    preferred_element_type=jnp.float32)
            + jnp.dot(ha0_scr[:, :].astype(jnp.bfloat16), ua_ref[0],
                      preferred_element_type=jnp.float32)
            + ba_ref[0], 0.0)
        sa1 = jnp.sum(ha1, axis=0, keepdims=True) / float(_A)
        c_scr[1:2, :] = (jnp.dot(sa1.astype(jnp.bfloat16), wv_ref[1],
                                 preferred_element_type=jnp.float32)
                         + bv_ref[1])

    @pl.when(p == 1)
    def _pass1():
        h2 = jnp.maximum(g1_scr[pl.ds(i * _B, _B), :] + c_scr[1:2, :], 0.0)
        out_ref[:, :] = (jnp.dot(h2.astype(jnp.bfloat16), wout_ref[:, :],
                                 preferred_element_type=jnp.float32)
                         + bout_ref[:, :])


def kernel(v, query_attrs, emb_table, W_in, b_in, Wa, Ua, ba, Wv, Uv, bv,
           W_out, b_out):
    qa2 = query_attrs.astype(jnp.int32).reshape(1, _A)
    bf = jnp.bfloat16
    W_in, Wa, Ua, Wv, Uv, W_out = (x.astype(bf) for x in
                                   (W_in, Wa, Ua, Wv, Uv, W_out))
    b_in2 = b_in.reshape(1, _HIDDEN)
    ba2 = ba.reshape(2, 1, _HIDDEN)
    bv2 = bv.reshape(2, 1, _HIDDEN)
    b_out2 = b_out.reshape(1, _NODE_DIM)

    const3 = lambda: pl.BlockSpec((2, _HIDDEN, _HIDDEN), lambda p, i: (0, 0, 0))
    cbias = lambda: pl.BlockSpec((2, 1, _HIDDEN), lambda p, i: (0, 0, 0))

    return pl.pallas_call(
        _body,
        grid=(2, _NB),
        in_specs=[
            pl.BlockSpec((1, _A), lambda p, i: (0, 0)),
            pl.BlockSpec((_B, _NODE_DIM),
                         lambda p, i: (jnp.where(p == 0, i, _NB - 1), 0)),
            pl.BlockSpec((_ATTR_DIM, _HIDDEN), lambda p, i: (0, 0)),
            pl.BlockSpec((_NODE_DIM, _HIDDEN), lambda p, i: (0, 0)),
            pl.BlockSpec((1, _HIDDEN), lambda p, i: (0, 0)),
            const3(),
            const3(),
            cbias(),
            const3(),
            const3(),
            cbias(),
            pl.BlockSpec((_HIDDEN, _NODE_DIM), lambda p, i: (0, 0)),
            pl.BlockSpec((1, _NODE_DIM), lambda p, i: (0, 0)),
        ],
        out_specs=pl.BlockSpec((_B, _NODE_DIM),
                               lambda p, i: (jnp.where(p == 1, i, 0), 0)),
        out_shape=jax.ShapeDtypeStruct((_N, _NODE_DIM), jnp.float32),
        scratch_shapes=[
            pltpu.VMEM((_N, _HIDDEN), jnp.float32),
            pltpu.VMEM((_A, _HIDDEN), jnp.float32),
            pltpu.VMEM((2, _HIDDEN), jnp.float32),
            pltpu.VMEM((1, _NODE_DIM), jnp.float32),
            pltpu.VMEM((_NODE_DIM, _HIDDEN), jnp.bfloat16),
        ],
    )(qa2, v, emb_table, W_in, b_in2, Wa, Ua, ba2, Wv, Uv, bv2, W_out, b_out2)


# speedup vs baseline: 1.7734x; 1.7734x over previous
import jax
import jax.numpy as jnp
from jax.experimental import pallas as pl
from jax.experimental.pallas import tpu as pltpu

_N = 10000
_A = 32
_NODE_DIM = 256
_ATTR_DIM = 512
_HIDDEN = 128
_B = 5000
_NB = _N // _B


def _body(qa_ref, v_ref, emb_ref, win_ref, bin_ref, wa_ref, ua_ref, ba_ref,
          wv_ref, uv_ref, bv_ref, wout_ref, bout_ref, out_ref,
          g1_scr, ha0_scr, c_scr, vsum_scr, m0_scr):
    p = pl.program_id(0)
    i = pl.program_id(1)

    @pl.when((p == 0) & (i == 0))
    def _init():
        oneT = (jax.lax.broadcasted_iota(jnp.int32, (_ATTR_DIM, _A), 0)
                == qa_ref[:, :]).astype(jnp.float32)
        ha0 = jax.lax.dot_general(oneT, emb_ref[:, :], (((0,), (0,)), ((), ())),
                                  preferred_element_type=jnp.float32)
        ha0_scr[:, :] = ha0
        sa0 = jnp.sum(ha0, axis=0, keepdims=True) / float(_A)
        c0 = (jnp.dot(sa0, wv_ref[0], preferred_element_type=jnp.float32)
              + bv_ref[0])
        m0_scr[:, :] = jnp.dot(win_ref[:, :], uv_ref[0],
                               preferred_element_type=jnp.float32)
        c_scr[0:1, :] = (jnp.dot(bin_ref[:, :], uv_ref[0],
                                 preferred_element_type=jnp.float32) + c0)
        vsum_scr[:, :] = jnp.zeros((1, _NODE_DIM), jnp.float32)

    @pl.when(p == 0)
    def _pass0():
        x = v_ref[:, :]
        vsum_scr[:, :] += jnp.sum(x, axis=0, keepdims=True)
        h1 = jnp.maximum(
            jnp.dot(x, m0_scr[:, :], preferred_element_type=jnp.float32)
            + c_scr[0:1, :], 0.0)
        g1_scr[pl.ds(i * _B, _B), :] = jnp.dot(
            h1, uv_ref[1], preferred_element_type=jnp.float32)

    @pl.when((p == 1) & (i == 0))
    def _mid():
        sv0 = (jnp.dot(vsum_scr[:, :] / float(_N), win_ref[:, :],
                       preferred_element_type=jnp.float32) + bin_ref[:, :])
        ha1 = jnp.maximum(
            jnp.dot(sv0, wa_ref[0], preferred_element_type=jnp.float32)
            + jnp.dot(ha0_scr[:, :], ua_ref[0],
                      preferred_element_type=jnp.float32)
            + ba_ref[0], 0.0)
        sa1 = jnp.sum(ha1, axis=0, keepdims=True) / float(_A)
        c_scr[1:2, :] = (jnp.dot(sa1, wv_ref[1],
                                 preferred_element_type=jnp.float32)
                         + bv_ref[1])

    @pl.when(p == 1)
    def _pass1():
        h2 = jnp.maximum(g1_scr[pl.ds(i * _B, _B), :] + c_scr[1:2, :], 0.0)
        out_ref[:, :] = (jnp.dot(h2, wout_ref[:, :],
                                 preferred_element_type=jnp.float32)
                         + bout_ref[:, :])


def kernel(v, query_attrs, emb_table, W_in, b_in, Wa, Ua, ba, Wv, Uv, bv,
           W_out, b_out):
    qa2 = query_attrs.astype(jnp.int32).reshape(1, _A)
    b_in2 = b_in.reshape(1, _HIDDEN)
    ba2 = ba.reshape(2, 1, _HIDDEN)
    bv2 = bv.reshape(2, 1, _HIDDEN)
    b_out2 = b_out.reshape(1, _NODE_DIM)

    const3 = lambda: pl.BlockSpec((2, _HIDDEN, _HIDDEN), lambda p, i: (0, 0, 0))
    cbias = lambda: pl.BlockSpec((2, 1, _HIDDEN), lambda p, i: (0, 0, 0))

    return pl.pallas_call(
        _body,
        grid=(2, _NB),
        in_specs=[
            pl.BlockSpec((1, _A), lambda p, i: (0, 0)),
            pl.BlockSpec((_B, _NODE_DIM),
                         lambda p, i: (jnp.where(p == 0, i, _NB - 1), 0)),
            pl.BlockSpec((_ATTR_DIM, _HIDDEN), lambda p, i: (0, 0)),
            pl.BlockSpec((_NODE_DIM, _HIDDEN), lambda p, i: (0, 0)),
            pl.BlockSpec((1, _HIDDEN), lambda p, i: (0, 0)),
            const3(),
            const3(),
            cbias(),
            const3(),
            const3(),
            cbias(),
            pl.BlockSpec((_HIDDEN, _NODE_DIM), lambda p, i: (0, 0)),
            pl.BlockSpec((1, _NODE_DIM), lambda p, i: (0, 0)),
        ],
        out_specs=pl.BlockSpec((_B, _NODE_DIM),
                               lambda p, i: (jnp.where(p == 1, i, 0), 0)),
        out_shape=jax.ShapeDtypeStruct((_N, _NODE_DIM), jnp.float32),
        scratch_shapes=[
            pltpu.VMEM((_N, _HIDDEN), jnp.float32),
            pltpu.VMEM((_A, _HIDDEN), jnp.float32),
            pltpu.VMEM((2, _HIDDEN), jnp.float32),
            pltpu.VMEM((1, _NODE_DIM), jnp.float32),
            pltpu.VMEM((_NODE_DIM, _HIDDEN), jnp.float32),
        ],
    )(qa2, v, emb_table, W_in, b_in2, Wa, Ua, ba2, Wv, Uv, bv2, W_out, b_out2)
